# baseline (speedup 1.0000x reference)
import functools

import jax
import jax.numpy as jnp
from jax import lax
from jax.experimental import pallas as pl
from jax.experimental.pallas import tpu as pltpu
from jax.experimental.pallas import tpu_sc as plsc

_ANCHOR_WH = (11.5, 22.9, 45.8)
_STRIDE = 8.0

_NC, _NS, _L = 2, 16, 16
_NW = _NC * _NS

_G = 76
_S = _G * _G
_C = 10
_ROW = 3 * _S


def _sig(v):
    return 1.0 / (1.0 + jnp.exp(-v))


@functools.lru_cache(maxsize=None)
def _build_decode():
    mesh = plsc.VectorSubcoreMesh(core_axis_name="c", subcore_axis_name="s",
                                  num_cores=_NC, num_subcores=_NS)
    return pl.kernel(
        _decode_body,
        out_type=jax.ShapeDtypeStruct((_C, 64, _ROW), jnp.float32),
        mesh=mesh,
        compiler_params=pltpu.CompilerParams(use_tc_tiling_on_sc=True),
        scratch_types=[
            pltpu.VMEM((3, _G, _G), jnp.float32),
            pltpu.VMEM((_ROW,), jnp.float32),
            pltpu.SemaphoreType.DMA,
        ],
    )


def _decode_body(x_hbm, out_hbm, in_v, out_v, sem):
    wid = lax.axis_index("s") * _NC + lax.axis_index("c")
    iota = lax.iota(jnp.int32, _L)
    iota_f = iota.astype(jnp.float32)
    col_starts = (0, 16, 32, 48, _G - _L)

    for c in range(_C):
        def _task(jb, carry, c=c):
            b = wid + _NW * jb
            hs = [pltpu.async_copy(x_hbm.at[b * 30 + a * _C + c],
                                   in_v.at[a], sem)
                  for a in range(3)]
            for h in hs:
                h.wait()

            for a in range(3):
                anchor = _ANCHOR_WH[a]

                @plsc.parallel_loop(0, _G, unroll=1)
                def _body(r, a=a, c=c, anchor=anchor):
                    base = a * _S + r * _G
                    gy8 = r.astype(jnp.float32) * _STRIDE
                    for col0 in col_starts:
                        v = in_v[a, r, pl.ds(col0, _L)]
                        if c == 0:
                            r_val = _sig(v) * _STRIDE + (iota_f + col0) * _STRIDE
                        elif c == 1:
                            r_val = _sig(v) * _STRIDE + gy8
                        elif c in (2, 3):
                            r_val = jnp.exp(v) * anchor
                        elif c in (4, 5):
                            r_val = v
                        else:
                            r_val = _sig(v)
                        out_v[pl.ds(base + col0, _L)] = r_val

            pltpu.sync_copy(out_v, out_hbm.at[c, b])
            return carry

        lax.fori_loop(0, 2, _task, None)


def kernel(x):
    nB = x.shape[0]
    out = _build_decode()(x.reshape(nB * 30, _G, _G))
    return out.transpose(1, 2, 0)


# speedup vs baseline: 1.3084x; 1.3084x over previous
import functools

import jax
import jax.numpy as jnp
from jax import lax
from jax.experimental import pallas as pl
from jax.experimental.pallas import tpu as pltpu
from jax.experimental.pallas import tpu_sc as plsc

_ANCHOR_WH = (11.5, 22.9, 45.8)
_STRIDE = 8.0

_NC, _NS, _L = 2, 16, 16
_NW = _NC * _NS

_G = 76
_S = _G * _G
_C = 10
_ROW = 3 * _S

_TASKS = [(c, jb) for c in range(_C) for jb in range(2)]


def _sig(v):
    return 1.0 / (1.0 + jnp.exp(-v))


@functools.lru_cache(maxsize=None)
def _build_decode():
    mesh = plsc.VectorSubcoreMesh(core_axis_name="c", subcore_axis_name="s",
                                  num_cores=_NC, num_subcores=_NS)
    return pl.kernel(
        _decode_body,
        out_type=jax.ShapeDtypeStruct((_C, 64, _ROW), jnp.float32),
        mesh=mesh,
        compiler_params=pltpu.CompilerParams(use_tc_tiling_on_sc=True),
        scratch_types=[
            pltpu.VMEM((2, 3 * _G, _G), jnp.float32),
            pltpu.VMEM((_ROW,), jnp.float32),
            pltpu.VMEM((_ROW,), jnp.float32),
            pltpu.SemaphoreType.DMA,
            pltpu.SemaphoreType.DMA,
            pltpu.SemaphoreType.DMA,
            pltpu.SemaphoreType.DMA,
        ],
    )


def _decode_body(x_hbm, out_hbm, in_v, ov0, ov1, si0, si1, so0, so1):
    wid = lax.axis_index("s") * _NC + lax.axis_index("c")
    iota_f = lax.iota(jnp.int32, _L).astype(jnp.float32)
    col_starts = (0, 16, 32, 48, _G - _L)
    sin = (si0, si1)
    sout = (so0, so1)
    ovs = (ov0, ov1)

    def fire_in(t):
        c, jb = _TASKS[t]
        b = wid + _NW * jb
        buf = t % 2
        return [pltpu.async_copy(x_hbm.at[b * 30 + a * _C + c],
                                 in_v.at[buf, pl.ds(a * _G, _G), :], sin[buf])
                for a in range(3)]

    hin = {0: fire_in(0)}
    hout = {}
    for t, (c, jb) in enumerate(_TASKS):
        b = wid + _NW * jb
        buf = t % 2
        for h in hin.pop(t):
            h.wait()
        if t + 1 < len(_TASKS):
            hin[t + 1] = fire_in(t + 1)
        if t - 2 in hout:
            hout.pop(t - 2).wait()

        @plsc.parallel_loop(0, 3 * _G, unroll=1)
        def _body(ar, c=c, buf=buf):
            base = ar * _G
            if c == 1:
                na = (ar >= _G).astype(jnp.int32) + (ar >= 2 * _G).astype(jnp.int32)
                gy8 = (ar - na * _G).astype(jnp.float32) * _STRIDE
            elif c in (2, 3):
                anchor = jnp.where(ar < _G, _ANCHOR_WH[0],
                                   jnp.where(ar < 2 * _G, _ANCHOR_WH[1],
                                             _ANCHOR_WH[2]))
            for col0 in col_starts:
                v = in_v[buf, ar, pl.ds(col0, _L)]
                if c == 0:
                    r_val = _sig(v) * _STRIDE + (iota_f + col0) * _STRIDE
                elif c == 1:
                    r_val = _sig(v) * _STRIDE + gy8
                elif c in (2, 3):
                    r_val = jnp.exp(v) * anchor
                elif c in (4, 5):
                    r_val = v
                else:
                    r_val = _sig(v)
                ovs[buf][pl.ds(base + col0, _L)] = r_val

        hout[t] = pltpu.async_copy(ovs[buf], out_hbm.at[c, b], sout[buf])
    for h in hout.values():
        h.wait()


def kernel(x):
    nB = x.shape[0]
    out = _build_decode()(x.reshape(nB * 30, _G, _G))
    return out.transpose(1, 2, 0)


# speedup vs baseline: 2.2917x; 1.7516x over previous
import functools

import jax
import jax.numpy as jnp
from jax import lax
from jax.experimental import pallas as pl
from jax.experimental.pallas import tpu as pltpu
from jax.experimental.pallas import tpu_sc as plsc

_ANCHOR_WH = (11.5, 22.9, 45.8)
_STRIDE = 8.0

_NC, _NS, _L = 2, 16, 16
_NW = _NC * _NS

_G = 76
_S = _G * _G
_C = 10
_ROW = 3 * _S

_TASKS = [(c, jb) for c in range(_C) for jb in range(2)]


def _sig(v):
    return 1.0 / (1.0 + jnp.exp(-v))


@functools.lru_cache(maxsize=None)
def _build_decode():
    mesh = plsc.VectorSubcoreMesh(core_axis_name="c", subcore_axis_name="s",
                                  num_cores=_NC, num_subcores=_NS)
    return pl.kernel(
        _decode_body,
        out_type=jax.ShapeDtypeStruct((_C, 64, _ROW), jnp.float32),
        mesh=mesh,
        compiler_params=pltpu.CompilerParams(use_tc_tiling_on_sc=True),
        scratch_types=[
            pltpu.VMEM((2, 3 * _G, _G), jnp.float32),
            pltpu.VMEM((_ROW,), jnp.float32),
            pltpu.VMEM((_ROW,), jnp.float32),
            pltpu.SemaphoreType.DMA,
            pltpu.SemaphoreType.DMA,
            pltpu.SemaphoreType.DMA,
            pltpu.SemaphoreType.DMA,
        ],
    )


def _decode_body(x_hbm, out_hbm, in_v, ov0, ov1, si0, si1, so0, so1):
    wid = lax.axis_index("s") * _NC + lax.axis_index("c")
    iota_f = lax.iota(jnp.int32, _L).astype(jnp.float32)
    col_starts = (0, 16, 32, 48, _G - _L)
    sin = (si0, si1)
    sout = (so0, so1)
    ovs = (ov0, ov1)

    def fire_in(t):
        c, jb = _TASKS[t]
        b = wid + _NW * jb
        buf = t % 2
        return [pltpu.async_copy(x_hbm.at[a * _C + c, :, b, :],
                                 in_v.at[buf, pl.ds(a * _G, _G), :], sin[buf])
                for a in range(3)]

    hin = {0: fire_in(0)}
    hout = {}
    for t, (c, jb) in enumerate(_TASKS):
        b = wid + _NW * jb
        buf = t % 2
        for h in hin.pop(t):
            h.wait()
        if t + 1 < len(_TASKS):
            hin[t + 1] = fire_in(t + 1)
        if t - 2 in hout:
            hout.pop(t - 2).wait()

        @plsc.parallel_loop(0, 3 * _G, unroll=1)
        def _body(ar, c=c, buf=buf):
            base = ar * _G
            if c == 1:
                na = (ar >= _G).astype(jnp.int32) + (ar >= 2 * _G).astype(jnp.int32)
                gy8 = (ar - na * _G).astype(jnp.float32) * _STRIDE
            elif c in (2, 3):
                anchor = jnp.where(ar < _G, _ANCHOR_WH[0],
                                   jnp.where(ar < 2 * _G, _ANCHOR_WH[1],
                                             _ANCHOR_WH[2]))
            for col0 in col_starts:
                v = in_v[buf, ar, pl.ds(col0, _L)]
                if c == 0:
                    r_val = _sig(v) * _STRIDE + (iota_f + col0) * _STRIDE
                elif c == 1:
                    r_val = _sig(v) * _STRIDE + gy8
                elif c in (2, 3):
                    r_val = jnp.exp(v) * anchor
                elif c in (4, 5):
                    r_val = v
                else:
                    r_val = _sig(v)
                ovs[buf][pl.ds(base + col0, _L)] = r_val

        hout[t] = pltpu.async_copy(ovs[buf], out_hbm.at[c, b], sout[buf])
    for h in hout.values():
        h.wait()


def kernel(x):
    nB = x.shape[0]
    out = _build_decode()(x.transpose(1, 2, 0, 3))
    return out.transpose(1, 2, 0)


# speedup vs baseline: 2.3169x; 1.0110x over previous
import functools

import jax
import jax.numpy as jnp
from jax import lax
from jax.experimental import pallas as pl
from jax.experimental.pallas import tpu as pltpu
from jax.experimental.pallas import tpu_sc as plsc

_ANCHOR_WH = (11.5, 22.9, 45.8)
_STRIDE = 8.0

_NC, _NS, _L = 2, 16, 16
_NW = _NC * _NS

_G = 76
_S = _G * _G
_C = 10
_ROW = 3 * _S

_TASKS = [(c, jb) for c in range(_C) for jb in range(2)]


def _sig(v):
    return 1.0 / (1.0 + jnp.exp(-v))


@functools.lru_cache(maxsize=None)
def _build_decode():
    mesh = plsc.VectorSubcoreMesh(core_axis_name="c", subcore_axis_name="s",
                                  num_cores=_NC, num_subcores=_NS)
    return pl.kernel(
        _decode_body,
        out_type=jax.ShapeDtypeStruct((_C, 64, _ROW), jnp.float32),
        mesh=mesh,
        compiler_params=pltpu.CompilerParams(use_tc_tiling_on_sc=True),
        scratch_types=[
            pltpu.VMEM((2, 3 * _G, _G), jnp.float32),
            pltpu.VMEM((_ROW,), jnp.float32),
            pltpu.VMEM((_ROW,), jnp.float32),
            pltpu.SemaphoreType.DMA,
            pltpu.SemaphoreType.DMA,
            pltpu.SemaphoreType.DMA,
            pltpu.SemaphoreType.DMA,
        ],
    )


def _decode_body(x_hbm, out_hbm, in_v, ov0, ov1, si0, si1, so0, so1):
    wid = lax.axis_index("s") * _NC + lax.axis_index("c")
    iota_f = lax.iota(jnp.int32, _L).astype(jnp.float32)
    col_starts = (0, 16, 32, 48, _G - _L)
    sin = (si0, si1)
    sout = (so0, so1)
    ovs = (ov0, ov1)

    def fire_in(t):
        c, jb = _TASKS[t]
        b = wid + _NW * jb
        buf = t % 2
        return [pltpu.async_copy(x_hbm.at[a * _C + c, :, b, :],
                                 in_v.at[buf, pl.ds(a * _G, _G), :], sin[buf])
                for a in range(3)]

    hin = {0: fire_in(0)}
    hout = {}
    for t, (c, jb) in enumerate(_TASKS):
        b = wid + _NW * jb
        buf = t % 2
        for h in hin.pop(t):
            h.wait()
        if t + 1 < len(_TASKS):
            hin[t + 1] = fire_in(t + 1)
        if t - 2 in hout:
            hout.pop(t - 2).wait()

        @plsc.parallel_loop(0, 3 * _G, unroll=2)
        def _body(ar, c=c, buf=buf):
            base = ar * _G
            if c == 1:
                na = (ar >= _G).astype(jnp.int32) + (ar >= 2 * _G).astype(jnp.int32)
                gy8 = (ar - na * _G).astype(jnp.float32) * _STRIDE
            elif c in (2, 3):
                anchor = jnp.where(ar < _G, _ANCHOR_WH[0],
                                   jnp.where(ar < 2 * _G, _ANCHOR_WH[1],
                                             _ANCHOR_WH[2]))
            for col0 in col_starts:
                v = in_v[buf, ar, pl.ds(col0, _L)]
                if c == 0:
                    r_val = _sig(v) * _STRIDE + (iota_f + col0) * _STRIDE
                elif c == 1:
                    r_val = _sig(v) * _STRIDE + gy8
                elif c in (2, 3):
                    r_val = jnp.exp(v) * anchor
                elif c in (4, 5):
                    r_val = v
                else:
                    r_val = _sig(v)
                ovs[buf][pl.ds(base + col0, _L)] = r_val

        hout[t] = pltpu.async_copy(ovs[buf], out_hbm.at[c, b], sout[buf])
    for h in hout.values():
        h.wait()


def kernel(x):
    nB = x.shape[0]
    out = _build_decode()(x.transpose(1, 2, 0, 3))
    return out.transpose(1, 2, 0)


# speedup vs baseline: 2.3250x; 1.0035x over previous
import functools

import jax
import jax.numpy as jnp
from jax import lax
from jax.experimental import pallas as pl
from jax.experimental.pallas import tpu as pltpu
from jax.experimental.pallas import tpu_sc as plsc

_ANCHOR_WH = (11.5, 22.9, 45.8)
_STRIDE = 8.0

_NC, _NS, _L = 2, 16, 16
_NW = _NC * _NS

_G = 76
_S = _G * _G
_C = 10
_ROW = 3 * _S

_TASKS = [(c, jb) for c in range(_C) for jb in range(2)]


def _sig(v):
    return 1.0 / (1.0 + jnp.exp(-v))


@functools.lru_cache(maxsize=None)
def _build_decode():
    mesh = plsc.VectorSubcoreMesh(core_axis_name="c", subcore_axis_name="s",
                                  num_cores=_NC, num_subcores=_NS)
    return pl.kernel(
        _decode_body,
        out_type=jax.ShapeDtypeStruct((_C, 64, _ROW), jnp.float32),
        mesh=mesh,
        compiler_params=pltpu.CompilerParams(use_tc_tiling_on_sc=True),
        scratch_types=[
            pltpu.VMEM((2, 3 * _G, _G), jnp.float32),
            pltpu.VMEM((_ROW,), jnp.float32),
            pltpu.VMEM((_ROW,), jnp.float32),
            pltpu.SemaphoreType.DMA,
            pltpu.SemaphoreType.DMA,
            pltpu.SemaphoreType.DMA,
            pltpu.SemaphoreType.DMA,
        ],
    )


def _decode_body(x_hbm, out_hbm, in_v, ov0, ov1, si0, si1, so0, so1):
    wid = lax.axis_index("s") * _NC + lax.axis_index("c")
    iota_f = lax.iota(jnp.int32, _L).astype(jnp.float32)
    col_starts = (0, 16, 32, 48, _G - _L)
    sin = (si0, si1)
    sout = (so0, so1)
    ovs = (ov0, ov1)

    def fire_in(t):
        c, jb = _TASKS[t]
        b = wid + _NW * jb
        buf = t % 2
        return [pltpu.async_copy(x_hbm.at[a * _C + c, :, b, :],
                                 in_v.at[buf, pl.ds(a * _G, _G), :], sin[buf])
                for a in range(3)]

    hin = {0: fire_in(0)}
    hout = {}
    for t, (c, jb) in enumerate(_TASKS):
        b = wid + _NW * jb
        buf = t % 2
        for h in hin.pop(t):
            h.wait()
        if t + 1 < len(_TASKS):
            hin[t + 1] = fire_in(t + 1)
        if t - 2 in hout:
            hout.pop(t - 2).wait()

        @plsc.parallel_loop(0, 3 * _G, unroll=4)
        def _body(ar, c=c, buf=buf):
            base = ar * _G
            if c == 1:
                na = (ar >= _G).astype(jnp.int32) + (ar >= 2 * _G).astype(jnp.int32)
                gy8 = (ar - na * _G).astype(jnp.float32) * _STRIDE
            elif c in (2, 3):
                anchor = jnp.where(ar < _G, _ANCHOR_WH[0],
                                   jnp.where(ar < 2 * _G, _ANCHOR_WH[1],
                                             _ANCHOR_WH[2]))
            for col0 in col_starts:
                v = in_v[buf, ar, pl.ds(col0, _L)]
                if c == 0:
                    r_val = _sig(v) * _STRIDE + (iota_f + col0) * _STRIDE
                elif c == 1:
                    r_val = _sig(v) * _STRIDE + gy8
                elif c in (2, 3):
                    r_val = jnp.exp(v) * anchor
                elif c in (4, 5):
                    r_val = v
                else:
                    r_val = _sig(v)
                ovs[buf][pl.ds(base + col0, _L)] = r_val

        hout[t] = pltpu.async_copy(ovs[buf], out_hbm.at[c, b], sout[buf])
    for h in hout.values():
        h.wait()


def kernel(x):
    nB = x.shape[0]
    out = _build_decode()(x.transpose(1, 2, 0, 3))
    return out.transpose(1, 2, 0)
